# baseline (speedup 1.0000x reference)
import jax
import jax.numpy as jnp
from jax.experimental import pallas as pl
from jax.experimental.pallas import tpu as pltpu

_IN_RES = 512.0
_IOU_T = 0.75
_SCORE_T = 0.25
_MASK_T = 0.5
_MAXD = 300

_N = 5376
_B = 256
_NB = _N // _B
_SLOTS = 320
_PBLK = 4096
_NPIX = 128 * 128
_NCH = _N // 128


def _nms_select_kernel(boxes_ref, info_ref, payload_ref, out_ref):
    f32 = jnp.float32
    y1a = info_ref[0:1, :]
    x1a = info_ref[1:2, :]
    y2a = info_ref[2:3, :]
    x2a = info_ref[3:4, :]
    area_a = (y2a - y1a) * (x2a - x1a)
    scores_row = info_ref[4:5, :]

    keep_blocks = [
        (scores_row[:, bi * _B:(bi + 1) * _B] >= _SCORE_T).astype(f32)
        for bi in range(_NB)
    ]

    r_i = jax.lax.broadcasted_iota(jnp.int32, (_B, _B), 0)
    c_i = jax.lax.broadcasted_iota(jnp.int32, (_B, _B), 1)
    upper = (r_i < c_i).astype(f32)

    for bi in range(_NB):
        base = bi * _B
        rb = boxes_ref[base:base + _B, :]
        y1r = rb[:, 0:1]
        x1r = rb[:, 1:2]
        y2r = rb[:, 2:3]
        x2r = rb[:, 3:4]
        area_r = (y2r - y1r) * (x2r - x1r)

        def _sup_tile(c0):
            iy1 = jnp.maximum(y1r, y1a[:, c0:c0 + _B])
            ix1 = jnp.maximum(x1r, x1a[:, c0:c0 + _B])
            iy2 = jnp.minimum(y2r, y2a[:, c0:c0 + _B])
            ix2 = jnp.minimum(x2r, x2a[:, c0:c0 + _B])
            inter = jnp.maximum(iy2 - iy1, 0.0) * jnp.maximum(ix2 - ix1, 0.0)
            union = area_r + area_a[:, c0:c0 + _B] - inter + 1e-9
            s = (4.0 * inter - 2.0 * union) - union
            return (s > union * 1.1920928955078125e-07).astype(f32)

        s_mat = _sup_tile(base) * upper
        k0 = keep_blocks[bi]

        def _cond(carry):
            return carry[1]

        def _body(carry):
            k, _ = carry
            sup = jnp.dot(k, s_mat, preferred_element_type=f32)
            k_new = jnp.where(sup > 0.5, 0.0, k0)
            return k_new, jnp.any(k_new != k)

        kf, _ = jax.lax.while_loop(_cond, _body, (k0, jnp.bool_(True)))
        keep_blocks[bi] = kf

        for bj in range(bi + 1, _NB):
            o_tile = _sup_tile(bj * _B)
            sup = jnp.dot(kf, o_tile, preferred_element_type=f32)
            keep_blocks[bj] = jnp.where(sup > 0.5, 0.0, keep_blocks[bj])

    keep = jnp.concatenate(keep_blocks, axis=1)

    tr_r = jax.lax.broadcasted_iota(jnp.int32, (128, 128), 0)
    tr_c = jax.lax.broadcasted_iota(jnp.int32, (128, 128), 1)
    tri = (tr_r <= tr_c).astype(f32)
    slot_i = jax.lax.broadcasted_iota(jnp.int32, (_SLOTS, 128), 0)
    slot_f = slot_i.astype(f32)
    slot_ok = slot_i < _MAXD

    withins = []
    for ch in range(_NCH):
        kc = keep[:, ch * 128:(ch + 1) * 128]
        withins.append(jnp.dot(kc, tri, preferred_element_type=f32))
    totals = jnp.concatenate([w[:, 127:128] for w in withins], axis=1)
    pr_r = jax.lax.broadcasted_iota(jnp.int32, (_NCH, _NCH), 0)
    pr_c = jax.lax.broadcasted_iota(jnp.int32, (_NCH, _NCH), 1)
    pref = jnp.dot(totals, (pr_r < pr_c).astype(f32),
                   preferred_element_type=f32)

    pieces = []
    for ch in range(_NCH):
        kc = keep[:, ch * 128:(ch + 1) * 128]
        rank = withins[ch] + pref[:, ch:ch + 1]
        hit = (rank - 1.0 == slot_f) & (kc > 0.0) & slot_ok
        pieces.append(jnp.where(hit, 1.0, 0.0))
    p_mat = jnp.concatenate(pieces, axis=1)

    out_ref[...] = jnp.dot(p_mat, payload_ref[...],
                           precision=jax.lax.Precision.HIGHEST,
                           preferred_element_type=f32)


def _mask_kernel(proto_ref, selt_ref, out_ref):
    f32 = jnp.float32
    c = pl.program_id(0)
    coeff_t = selt_ref[8:40, :]
    z = jnp.dot(proto_ref[...], coeff_t, preferred_element_type=f32)
    m = jax.nn.sigmoid(z)
    m = jnp.where(m >= _MASK_T, m, 0.0)

    ratio_inv = jnp.asarray(128.0 / _IN_RES, f32)
    bb0 = selt_ref[0:1, :] * ratio_inv
    bb1 = selt_ref[1:2, :] * ratio_inv
    bb2 = selt_ref[2:3, :] * ratio_inv
    bb3 = selt_ref[3:4, :] * ratio_inv
    vf = selt_ref[4:5, :] > 0.0

    g = c * _PBLK + jax.lax.broadcasted_iota(jnp.int32, (_PBLK, 1), 0)
    rows_f = (g // 128).astype(f32)
    cols_f = (g % 128).astype(f32)
    ok = ((bb0 <= rows_f) & (rows_f <= bb2) &
          (bb1 <= cols_f) & (cols_f <= bb3) & vf)
    t = jnp.where(ok, m, 0.0)
    out_ref[...] = jnp.sum(t, axis=1, keepdims=True)[None]


def kernel(boxes, proto):
    f32 = boxes.dtype
    b = boxes[0]
    box = b[:4]
    prob = b[4:8]
    coeff = b[8:]
    scores = jnp.max(prob, axis=0)
    cls = jnp.argmax(prob, axis=0)

    bt = box.T
    xyxy = jnp.concatenate([bt[:, :2] - bt[:, 2:] / 2,
                            bt[:, :2] + bt[:, 2:] / 2], axis=-1)
    b_nms = xyxy[:, jnp.array([1, 0, 3, 2])]

    order = jnp.argsort(-scores)
    payload = jnp.concatenate(
        [b_nms, scores[:, None], cls.astype(f32)[:, None],
         jnp.zeros((_N, 2), f32), coeff.T], axis=1)
    payload_s = payload[order]
    boxes_s = payload_s[:, :4]
    info_st = payload_s[:, :8].T

    sel = pl.pallas_call(
        _nms_select_kernel,
        out_shape=jax.ShapeDtypeStruct((_SLOTS, 40), f32),
        compiler_params=pltpu.CompilerParams(
            vmem_limit_bytes=48 * 1024 * 1024),
    )(boxes_s, info_st, payload_s)

    proto_flat = proto[0].reshape(_NPIX, 32)
    sel_t = sel.T
    masks_flat = pl.pallas_call(
        _mask_kernel,
        grid=(_NPIX // _PBLK,),
        in_specs=[
            pl.BlockSpec((_PBLK, 32), lambda c: (c, 0)),
            pl.BlockSpec((40, _SLOTS), lambda c: (0, 0)),
        ],
        out_specs=pl.BlockSpec((1, _PBLK, 1), lambda c: (c, 0, 0)),
        out_shape=jax.ShapeDtypeStruct((_NPIX // _PBLK, _PBLK, 1), f32),
        compiler_params=pltpu.CompilerParams(
            dimension_semantics=("parallel",),
            vmem_limit_bytes=48 * 1024 * 1024),
    )(proto_flat, sel_t)
    masks = masks_flat.reshape(128, 128)

    return (sel[:_MAXD, 4:5], sel[:_MAXD, 5:6], sel[:_MAXD, 0:4], masks)


# speedup vs baseline: 1.1155x; 1.1155x over previous
import jax
import jax.numpy as jnp
from jax.experimental import pallas as pl
from jax.experimental.pallas import tpu as pltpu

_IN_RES = 512.0
_IOU_T = 0.75
_SCORE_T = 0.25
_MASK_T = 0.5
_MAXD = 300

_N = 5376
_B = 256
_NB = _N // _B
_SLOTS = 320
_PBLK = 4096
_NPIX = 128 * 128
_NCH = _N // 128


def _nms_select_kernel(boxes_ref, info_ref, payload_ref, out_ref):
    f32 = jnp.float32
    y1a = info_ref[0:1, :]
    x1a = info_ref[1:2, :]
    y2a = info_ref[2:3, :]
    x2a = info_ref[3:4, :]
    area_a = (y2a - y1a) * (x2a - x1a)
    scores_row = info_ref[4:5, :]

    keep_blocks = [
        (scores_row[:, bi * _B:(bi + 1) * _B] >= _SCORE_T).astype(f32)
        for bi in range(_NB)
    ]

    r_i = jax.lax.broadcasted_iota(jnp.int32, (_B, _B), 0)
    c_i = jax.lax.broadcasted_iota(jnp.int32, (_B, _B), 1)
    upper = (r_i < c_i).astype(f32)

    for bi in range(_NB):
        base = bi * _B
        rb = boxes_ref[base:base + _B, :]
        y1r = rb[:, 0:1]
        x1r = rb[:, 1:2]
        y2r = rb[:, 2:3]
        x2r = rb[:, 3:4]
        area_r = (y2r - y1r) * (x2r - x1r)

        def _sup_tile(c0):
            iy1 = jnp.maximum(y1r, y1a[:, c0:c0 + _B])
            ix1 = jnp.maximum(x1r, x1a[:, c0:c0 + _B])
            iy2 = jnp.minimum(y2r, y2a[:, c0:c0 + _B])
            ix2 = jnp.minimum(x2r, x2a[:, c0:c0 + _B])
            inter = jnp.maximum(iy2 - iy1, 0.0) * jnp.maximum(ix2 - ix1, 0.0)
            union = area_r + area_a[:, c0:c0 + _B] - inter + 1e-9
            s = (4.0 * inter - 2.0 * union) - union
            return (s > union * 1.1920928955078125e-07).astype(f32)

        s_mat = _sup_tile(base) * upper
        k0 = keep_blocks[bi]

        def _step(k):
            sup = jnp.dot(k, s_mat, preferred_element_type=f32)
            return jnp.where(sup > 0.5, 0.0, k0)

        def _cond(carry):
            return carry[1]

        def _body(carry):
            k, _ = carry
            k_new = _step(_step(k))
            return k_new, jnp.any(k_new != k)

        kf, _ = jax.lax.while_loop(_cond, _body, (_step(k0), jnp.bool_(True)))
        keep_blocks[bi] = kf

        for bj in range(bi + 1, _NB):
            o_tile = _sup_tile(bj * _B)
            sup = jnp.dot(kf, o_tile, preferred_element_type=f32)
            keep_blocks[bj] = jnp.where(sup > 0.5, 0.0, keep_blocks[bj])

    keep = jnp.concatenate(keep_blocks, axis=1)

    tr_r = jax.lax.broadcasted_iota(jnp.int32, (128, 128), 0)
    tr_c = jax.lax.broadcasted_iota(jnp.int32, (128, 128), 1)
    tri = (tr_r <= tr_c).astype(f32)
    slot_i = jax.lax.broadcasted_iota(jnp.int32, (_SLOTS, 128), 0)
    slot_f = slot_i.astype(f32)
    slot_ok = slot_i < _MAXD

    withins = []
    for ch in range(_NCH):
        kc = keep[:, ch * 128:(ch + 1) * 128]
        withins.append(jnp.dot(kc, tri, preferred_element_type=f32))
    totals = jnp.concatenate([w[:, 127:128] for w in withins], axis=1)
    pr_r = jax.lax.broadcasted_iota(jnp.int32, (_NCH, _NCH), 0)
    pr_c = jax.lax.broadcasted_iota(jnp.int32, (_NCH, _NCH), 1)
    pref = jnp.dot(totals, (pr_r < pr_c).astype(f32),
                   preferred_element_type=f32)

    pieces = []
    for ch in range(_NCH):
        kc = keep[:, ch * 128:(ch + 1) * 128]
        rank = withins[ch] + pref[:, ch:ch + 1]
        hit = (rank - 1.0 == slot_f) & (kc > 0.0) & slot_ok
        pieces.append(jnp.where(hit, 1.0, 0.0))
    p_mat = jnp.concatenate(pieces, axis=1)

    out_ref[...] = jnp.dot(p_mat, payload_ref[...],
                           precision=jax.lax.Precision.HIGHEST,
                           preferred_element_type=f32)


def _mask_kernel(proto_ref, sel_ref, out_ref):
    f32 = jnp.float32
    c = pl.program_id(0)
    coeff = sel_ref[:, 8:40]
    z = jnp.dot(coeff, proto_ref[...], preferred_element_type=f32)
    m = jax.nn.sigmoid(z)
    m = jnp.where(m >= _MASK_T, m, 0.0)

    ratio_inv = jnp.asarray(128.0 / _IN_RES, f32)
    bb0 = sel_ref[:, 0:1] * ratio_inv
    bb1 = sel_ref[:, 1:2] * ratio_inv
    bb2 = sel_ref[:, 2:3] * ratio_inv
    bb3 = sel_ref[:, 3:4] * ratio_inv
    vf = sel_ref[:, 4:5] > 0.0

    g = c * _PBLK + jax.lax.broadcasted_iota(jnp.int32, (1, _PBLK), 1)
    rows_f = (g // 128).astype(f32)
    cols_f = (g % 128).astype(f32)
    ok = ((bb0 <= rows_f) & (rows_f <= bb2) &
          (bb1 <= cols_f) & (cols_f <= bb3) & vf)
    t = jnp.where(ok, m, 0.0)
    out_ref[...] = jnp.sum(t, axis=0, keepdims=True)


def kernel(boxes, proto):
    f32 = boxes.dtype
    b = boxes[0]
    box = b[:4]
    prob = b[4:8]
    coeff = b[8:]
    scores = jnp.max(prob, axis=0)
    cls = jnp.argmax(prob, axis=0)

    bt = box.T
    xyxy = jnp.concatenate([bt[:, :2] - bt[:, 2:] / 2,
                            bt[:, :2] + bt[:, 2:] / 2], axis=-1)
    b_nms = xyxy[:, jnp.array([1, 0, 3, 2])]

    order = jnp.argsort(-scores)
    payload = jnp.concatenate(
        [b_nms, scores[:, None], cls.astype(f32)[:, None],
         jnp.zeros((_N, 2), f32), coeff.T], axis=1)
    payload_s = payload[order]
    boxes_s = payload_s[:, :4]
    info_st = payload_s[:, :8].T

    sel = pl.pallas_call(
        _nms_select_kernel,
        out_shape=jax.ShapeDtypeStruct((_SLOTS, 40), f32),
        compiler_params=pltpu.CompilerParams(
            vmem_limit_bytes=48 * 1024 * 1024),
    )(boxes_s, info_st, payload_s)

    proto_t = proto[0].reshape(_NPIX, 32).T
    masks_flat = pl.pallas_call(
        _mask_kernel,
        grid=(_NPIX // _PBLK,),
        in_specs=[
            pl.BlockSpec((32, _PBLK), lambda c: (0, c)),
            pl.BlockSpec((_SLOTS, 40), lambda c: (0, 0)),
        ],
        out_specs=pl.BlockSpec((1, _PBLK), lambda c: (0, c)),
        out_shape=jax.ShapeDtypeStruct((1, _NPIX), f32),
        compiler_params=pltpu.CompilerParams(
            dimension_semantics=("parallel",),
            vmem_limit_bytes=48 * 1024 * 1024),
    )(proto_t, sel)
    masks = masks_flat.reshape(128, 128)

    return (sel[:_MAXD, 4:5], sel[:_MAXD, 5:6], sel[:_MAXD, 0:4], masks)


# speedup vs baseline: 1.1205x; 1.0045x over previous
import jax
import jax.numpy as jnp
from jax.experimental import pallas as pl
from jax.experimental.pallas import tpu as pltpu

_IN_RES = 512.0
_IOU_T = 0.75
_SCORE_T = 0.25
_MASK_T = 0.5
_MAXD = 300

_N = 5376
_B = 256
_NB = _N // _B
_SLOTS = 320
_PBLK = 4096
_NPIX = 128 * 128
_NCH = _N // 128


def _nms_select_kernel(payload_ref, out_ref):
    f32 = jnp.float32
    info = payload_ref[:, 0:8].T
    y1a = info[0:1, :]
    x1a = info[1:2, :]
    y2a = info[2:3, :]
    x2a = info[3:4, :]
    area_a = (y2a - y1a) * (x2a - x1a)
    scores_row = info[4:5, :]

    keep_blocks = [
        (scores_row[:, bi * _B:(bi + 1) * _B] >= _SCORE_T).astype(f32)
        for bi in range(_NB)
    ]

    r_i = jax.lax.broadcasted_iota(jnp.int32, (_B, _B), 0)
    c_i = jax.lax.broadcasted_iota(jnp.int32, (_B, _B), 1)
    upper = (r_i < c_i).astype(f32)

    for bi in range(_NB):
        base = bi * _B
        rb = payload_ref[base:base + _B, 0:4]
        y1r = rb[:, 0:1]
        x1r = rb[:, 1:2]
        y2r = rb[:, 2:3]
        x2r = rb[:, 3:4]
        area_r = (y2r - y1r) * (x2r - x1r)

        def _sup_tile(c0):
            iy1 = jnp.maximum(y1r, y1a[:, c0:c0 + _B])
            ix1 = jnp.maximum(x1r, x1a[:, c0:c0 + _B])
            iy2 = jnp.minimum(y2r, y2a[:, c0:c0 + _B])
            ix2 = jnp.minimum(x2r, x2a[:, c0:c0 + _B])
            inter = jnp.maximum(iy2 - iy1, 0.0) * jnp.maximum(ix2 - ix1, 0.0)
            union = area_r + area_a[:, c0:c0 + _B] - inter + 1e-9
            s = (4.0 * inter - 2.0 * union) - union
            return (s > union * 1.1920928955078125e-07).astype(f32)

        s_mat = _sup_tile(base) * upper
        k0 = keep_blocks[bi]

        def _cond(carry):
            return carry[1]

        def _body(carry):
            k, _ = carry
            sup = jnp.dot(k, s_mat, preferred_element_type=f32)
            k_new = jnp.where(sup > 0.5, 0.0, k0)
            return k_new, jnp.any(k_new != k)

        kf, _ = jax.lax.while_loop(_cond, _body, (k0, jnp.bool_(True)))
        keep_blocks[bi] = kf

        for bj in range(bi + 1, _NB):
            o_tile = _sup_tile(bj * _B)
            sup = jnp.dot(kf, o_tile, preferred_element_type=f32)
            keep_blocks[bj] = jnp.where(sup > 0.5, 0.0, keep_blocks[bj])

    keep = jnp.concatenate(keep_blocks, axis=1)

    tr_r = jax.lax.broadcasted_iota(jnp.int32, (128, 128), 0)
    tr_c = jax.lax.broadcasted_iota(jnp.int32, (128, 128), 1)
    tri = (tr_r <= tr_c).astype(f32)
    slot_i = jax.lax.broadcasted_iota(jnp.int32, (_SLOTS, 128), 0)
    slot_f = slot_i.astype(f32)
    slot_ok = slot_i < _MAXD

    withins = []
    for ch in range(_NCH):
        kc = keep[:, ch * 128:(ch + 1) * 128]
        withins.append(jnp.dot(kc, tri, preferred_element_type=f32))
    totals = jnp.concatenate([w[:, 127:128] for w in withins], axis=1)
    pr_r = jax.lax.broadcasted_iota(jnp.int32, (_NCH, _NCH), 0)
    pr_c = jax.lax.broadcasted_iota(jnp.int32, (_NCH, _NCH), 1)
    pref = jnp.dot(totals, (pr_r < pr_c).astype(f32),
                   preferred_element_type=f32)

    pieces = []
    for ch in range(_NCH):
        kc = keep[:, ch * 128:(ch + 1) * 128]
        rank = withins[ch] + pref[:, ch:ch + 1]
        hit = (rank - 1.0 == slot_f) & (kc > 0.0) & slot_ok
        pieces.append(jnp.where(hit, 1.0, 0.0))
    p_mat = jnp.concatenate(pieces, axis=1)

    out_ref[...] = jnp.dot(p_mat, payload_ref[...],
                           precision=jax.lax.Precision.HIGHEST,
                           preferred_element_type=f32)


def _mask_kernel(proto_ref, sel_ref, out_ref):
    f32 = jnp.float32
    c = pl.program_id(0)
    coeff = sel_ref[:, 8:40]
    z = jnp.dot(coeff, proto_ref[...], preferred_element_type=f32)
    m = jax.nn.sigmoid(z)
    m = jnp.where(m >= _MASK_T, m, 0.0)

    ratio_inv = jnp.asarray(128.0 / _IN_RES, f32)
    bb0 = sel_ref[:, 0:1] * ratio_inv
    bb1 = sel_ref[:, 1:2] * ratio_inv
    bb2 = sel_ref[:, 2:3] * ratio_inv
    bb3 = sel_ref[:, 3:4] * ratio_inv
    vf = sel_ref[:, 4:5] > 0.0

    g = c * _PBLK + jax.lax.broadcasted_iota(jnp.int32, (1, _PBLK), 1)
    rows_f = (g // 128).astype(f32)
    cols_f = (g % 128).astype(f32)
    ok = ((bb0 <= rows_f) & (rows_f <= bb2) &
          (bb1 <= cols_f) & (cols_f <= bb3) & vf)
    t = jnp.where(ok, m, 0.0)
    out_ref[...] = jnp.sum(t, axis=0, keepdims=True)


def kernel(boxes, proto):
    f32 = boxes.dtype
    b = boxes[0]
    box = b[:4]
    prob = b[4:8]
    coeff = b[8:]
    scores = jnp.max(prob, axis=0)
    cls = jnp.argmax(prob, axis=0)

    bt = box.T
    xyxy = jnp.concatenate([bt[:, :2] - bt[:, 2:] / 2,
                            bt[:, :2] + bt[:, 2:] / 2], axis=-1)
    b_nms = xyxy[:, jnp.array([1, 0, 3, 2])]

    order = jnp.argsort(-scores)
    payload = jnp.concatenate(
        [b_nms, scores[:, None], cls.astype(f32)[:, None],
         jnp.zeros((_N, 2), f32), coeff.T], axis=1)
    payload_s = payload[order]

    sel = pl.pallas_call(
        _nms_select_kernel,
        out_shape=jax.ShapeDtypeStruct((_SLOTS, 40), f32),
        compiler_params=pltpu.CompilerParams(
            vmem_limit_bytes=48 * 1024 * 1024),
    )(payload_s)

    proto_t = proto[0].reshape(_NPIX, 32).T
    masks_flat = pl.pallas_call(
        _mask_kernel,
        grid=(_NPIX // _PBLK,),
        in_specs=[
            pl.BlockSpec((32, _PBLK), lambda c: (0, c)),
            pl.BlockSpec((_SLOTS, 40), lambda c: (0, 0)),
        ],
        out_specs=pl.BlockSpec((1, _PBLK), lambda c: (0, c)),
        out_shape=jax.ShapeDtypeStruct((1, _NPIX), f32),
        compiler_params=pltpu.CompilerParams(
            dimension_semantics=("parallel",),
            vmem_limit_bytes=48 * 1024 * 1024),
    )(proto_t, sel)
    masks = masks_flat.reshape(128, 128)

    return (sel[:_MAXD, 4:5], sel[:_MAXD, 5:6], sel[:_MAXD, 0:4], masks)


# speedup vs baseline: 1.1297x; 1.0083x over previous
import jax
import jax.numpy as jnp
from jax.experimental import pallas as pl
from jax.experimental.pallas import tpu as pltpu

_IN_RES = 512.0
_IOU_T = 0.75
_SCORE_T = 0.25
_MASK_T = 0.5
_MAXD = 300

_N = 5376
_B = 256
_NB = _N // _B
_SLOTS = 320
_PBLK = 2048
_NPIX = 128 * 128
_NCH = _N // 128


def _nms_select_kernel(payload_ref, out_ref):
    f32 = jnp.float32
    info = payload_ref[:, 0:8].T
    y1a = info[0:1, :]
    x1a = info[1:2, :]
    y2a = info[2:3, :]
    x2a = info[3:4, :]
    area_a = (y2a - y1a) * (x2a - x1a)
    scores_row = info[4:5, :]

    keep_blocks = [
        (scores_row[:, bi * _B:(bi + 1) * _B] >= _SCORE_T).astype(f32)
        for bi in range(_NB)
    ]

    r_i = jax.lax.broadcasted_iota(jnp.int32, (_B, _B), 0)
    c_i = jax.lax.broadcasted_iota(jnp.int32, (_B, _B), 1)
    upper = (r_i < c_i).astype(f32)

    for bi in range(_NB):
        base = bi * _B
        rb = payload_ref[base:base + _B, 0:4]
        y1r = rb[:, 0:1]
        x1r = rb[:, 1:2]
        y2r = rb[:, 2:3]
        x2r = rb[:, 3:4]
        area_r = (y2r - y1r) * (x2r - x1r)

        def _sup_tile(c0):
            iy1 = jnp.maximum(y1r, y1a[:, c0:c0 + _B])
            ix1 = jnp.maximum(x1r, x1a[:, c0:c0 + _B])
            iy2 = jnp.minimum(y2r, y2a[:, c0:c0 + _B])
            ix2 = jnp.minimum(x2r, x2a[:, c0:c0 + _B])
            inter = jnp.maximum(iy2 - iy1, 0.0) * jnp.maximum(ix2 - ix1, 0.0)
            union = area_r + area_a[:, c0:c0 + _B] - inter + 1e-9
            s = (4.0 * inter - 2.0 * union) - union
            return (s > union * 1.1920928955078125e-07).astype(f32)

        s_mat = _sup_tile(base) * upper
        k0 = keep_blocks[bi]

        def _cond(carry):
            return carry[1]

        def _body(carry):
            k, _ = carry
            sup = jnp.dot(k, s_mat, preferred_element_type=f32)
            k_new = jnp.where(sup > 0.5, 0.0, k0)
            return k_new, jnp.any(k_new != k)

        kf, _ = jax.lax.while_loop(_cond, _body, (k0, jnp.bool_(True)))
        keep_blocks[bi] = kf

        for bj in range(bi + 1, _NB):
            o_tile = _sup_tile(bj * _B)
            sup = jnp.dot(kf, o_tile, preferred_element_type=f32)
            keep_blocks[bj] = jnp.where(sup > 0.5, 0.0, keep_blocks[bj])

    keep = jnp.concatenate(keep_blocks, axis=1)

    tr_r = jax.lax.broadcasted_iota(jnp.int32, (128, 128), 0)
    tr_c = jax.lax.broadcasted_iota(jnp.int32, (128, 128), 1)
    tri = (tr_r <= tr_c).astype(f32)
    slot_i = jax.lax.broadcasted_iota(jnp.int32, (_SLOTS, 128), 0)
    slot_f = slot_i.astype(f32)
    slot_ok = slot_i < _MAXD

    withins = []
    for ch in range(_NCH):
        kc = keep[:, ch * 128:(ch + 1) * 128]
        withins.append(jnp.dot(kc, tri, preferred_element_type=f32))
    totals = jnp.concatenate([w[:, 127:128] for w in withins], axis=1)
    pr_r = jax.lax.broadcasted_iota(jnp.int32, (_NCH, _NCH), 0)
    pr_c = jax.lax.broadcasted_iota(jnp.int32, (_NCH, _NCH), 1)
    pref = jnp.dot(totals, (pr_r < pr_c).astype(f32),
                   preferred_element_type=f32)

    pieces = []
    for ch in range(_NCH):
        kc = keep[:, ch * 128:(ch + 1) * 128]
        rank = withins[ch] + pref[:, ch:ch + 1]
        hit = (rank - 1.0 == slot_f) & (kc > 0.0) & slot_ok
        pieces.append(jnp.where(hit, 1.0, 0.0))
    p_mat = jnp.concatenate(pieces, axis=1)

    out_ref[...] = jnp.dot(p_mat, payload_ref[...],
                           precision=jax.lax.Precision.HIGHEST,
                           preferred_element_type=f32)


def _mask_kernel(proto_ref, sel_ref, out_ref):
    f32 = jnp.float32
    c = pl.program_id(0)
    coeff = sel_ref[:, 8:40]
    z = jnp.dot(coeff, proto_ref[...], preferred_element_type=f32)
    m = jax.nn.sigmoid(z)
    m = jnp.where(m >= _MASK_T, m, 0.0)

    ratio_inv = jnp.asarray(128.0 / _IN_RES, f32)
    bb0 = sel_ref[:, 0:1] * ratio_inv
    bb1 = sel_ref[:, 1:2] * ratio_inv
    bb2 = sel_ref[:, 2:3] * ratio_inv
    bb3 = sel_ref[:, 3:4] * ratio_inv
    vf = sel_ref[:, 4:5] > 0.0

    g = c * _PBLK + jax.lax.broadcasted_iota(jnp.int32, (1, _PBLK), 1)
    rows_f = (g // 128).astype(f32)
    cols_f = (g % 128).astype(f32)
    ok = ((bb0 <= rows_f) & (rows_f <= bb2) &
          (bb1 <= cols_f) & (cols_f <= bb3) & vf)
    t = jnp.where(ok, m, 0.0)
    out_ref[...] = jnp.sum(t, axis=0, keepdims=True)


def kernel(boxes, proto):
    f32 = boxes.dtype
    b = boxes[0]
    box = b[:4]
    prob = b[4:8]
    coeff = b[8:]
    scores = jnp.max(prob, axis=0)
    cls = jnp.argmax(prob, axis=0)

    bt = box.T
    xyxy = jnp.concatenate([bt[:, :2] - bt[:, 2:] / 2,
                            bt[:, :2] + bt[:, 2:] / 2], axis=-1)
    b_nms = xyxy[:, jnp.array([1, 0, 3, 2])]

    order = jnp.argsort(-scores)
    payload = jnp.concatenate(
        [b_nms, scores[:, None], cls.astype(f32)[:, None],
         jnp.zeros((_N, 2), f32), coeff.T], axis=1)
    payload_s = payload[order]

    sel = pl.pallas_call(
        _nms_select_kernel,
        out_shape=jax.ShapeDtypeStruct((_SLOTS, 40), f32),
        compiler_params=pltpu.CompilerParams(
            vmem_limit_bytes=48 * 1024 * 1024),
    )(payload_s)

    proto_t = proto[0].reshape(_NPIX, 32).T
    masks_flat = pl.pallas_call(
        _mask_kernel,
        grid=(_NPIX // _PBLK,),
        in_specs=[
            pl.BlockSpec((32, _PBLK), lambda c: (0, c)),
            pl.BlockSpec((_SLOTS, 40), lambda c: (0, 0)),
        ],
        out_specs=pl.BlockSpec((1, _PBLK), lambda c: (0, c)),
        out_shape=jax.ShapeDtypeStruct((1, _NPIX), f32),
        compiler_params=pltpu.CompilerParams(
            dimension_semantics=("parallel",),
            vmem_limit_bytes=48 * 1024 * 1024),
    )(proto_t, sel)
    masks = masks_flat.reshape(128, 128)

    return (sel[:_MAXD, 4:5], sel[:_MAXD, 5:6], sel[:_MAXD, 0:4], masks)


# speedup vs baseline: 1.1373x; 1.0067x over previous
import jax
import jax.numpy as jnp
from jax.experimental import pallas as pl
from jax.experimental.pallas import tpu as pltpu

_IN_RES = 512.0
_IOU_T = 0.75
_SCORE_T = 0.25
_MASK_T = 0.5
_MAXD = 300

_N = 5376
_B = 256
_NB = _N // _B
_SLOTS = 320
_PBLK = 4096
_NPIX = 128 * 128
_NCH = _N // 128


def _nms_select_kernel(payload_ref, out_ref):
    f32 = jnp.float32
    info = payload_ref[:, 0:8].T
    y1a = info[0:1, :]
    x1a = info[1:2, :]
    y2a = info[2:3, :]
    x2a = info[3:4, :]
    area_a = (y2a - y1a) * (x2a - x1a)
    scores_row = info[4:5, :]

    keep_blocks = [
        (scores_row[:, bi * _B:(bi + 1) * _B] >= _SCORE_T).astype(f32)
        for bi in range(_NB)
    ]

    r_i = jax.lax.broadcasted_iota(jnp.int32, (_B, _B), 0)
    c_i = jax.lax.broadcasted_iota(jnp.int32, (_B, _B), 1)
    upper = (r_i < c_i).astype(f32)

    for bi in range(_NB):
        base = bi * _B
        rb = payload_ref[base:base + _B, 0:4]
        y1r = rb[:, 0:1]
        x1r = rb[:, 1:2]
        y2r = rb[:, 2:3]
        x2r = rb[:, 3:4]
        area_r = (y2r - y1r) * (x2r - x1r)

        def _sup_tile(c0):
            iy1 = jnp.maximum(y1r, y1a[:, c0:c0 + _B])
            ix1 = jnp.maximum(x1r, x1a[:, c0:c0 + _B])
            iy2 = jnp.minimum(y2r, y2a[:, c0:c0 + _B])
            ix2 = jnp.minimum(x2r, x2a[:, c0:c0 + _B])
            inter = jnp.maximum(iy2 - iy1, 0.0) * jnp.maximum(ix2 - ix1, 0.0)
            union = area_r + area_a[:, c0:c0 + _B] - inter + 1e-9
            s = (4.0 * inter - 2.0 * union) - union
            return (s > union * 1.1920928955078125e-07).astype(f32)

        s_mat = _sup_tile(base) * upper
        k0 = keep_blocks[bi]

        def _cond(carry):
            return carry[1]

        def _body(carry):
            k, _ = carry
            sup = jnp.dot(k, s_mat, preferred_element_type=f32)
            k_new = jnp.where(sup > 0.5, 0.0, k0)
            return k_new, jnp.any(k_new != k)

        kf, _ = jax.lax.while_loop(_cond, _body, (k0, jnp.bool_(True)))
        keep_blocks[bi] = kf

        for bj in range(bi + 1, _NB):
            o_tile = _sup_tile(bj * _B)
            sup = jnp.dot(kf, o_tile, preferred_element_type=f32)
            keep_blocks[bj] = jnp.where(sup > 0.5, 0.0, keep_blocks[bj])

    keep = jnp.concatenate(keep_blocks, axis=1)

    tr_r = jax.lax.broadcasted_iota(jnp.int32, (128, 128), 0)
    tr_c = jax.lax.broadcasted_iota(jnp.int32, (128, 128), 1)
    tri = (tr_r <= tr_c).astype(f32)
    slot_i = jax.lax.broadcasted_iota(jnp.int32, (_SLOTS, 128), 0)
    slot_f = slot_i.astype(f32)
    slot_ok = slot_i < _MAXD

    withins = []
    for ch in range(_NCH):
        kc = keep[:, ch * 128:(ch + 1) * 128]
        withins.append(jnp.dot(kc, tri, preferred_element_type=f32))
    totals = jnp.concatenate([w[:, 127:128] for w in withins], axis=1)
    pr_r = jax.lax.broadcasted_iota(jnp.int32, (_NCH, _NCH), 0)
    pr_c = jax.lax.broadcasted_iota(jnp.int32, (_NCH, _NCH), 1)
    pref = jnp.dot(totals, (pr_r < pr_c).astype(f32),
                   preferred_element_type=f32)

    pieces = []
    for ch in range(_NCH):
        kc = keep[:, ch * 128:(ch + 1) * 128]
        rank = withins[ch] + pref[:, ch:ch + 1]
        hit = (rank - 1.0 == slot_f) & (kc > 0.0) & slot_ok
        pieces.append(jnp.where(hit, 1.0, 0.0))
    p_mat = jnp.concatenate(pieces, axis=1)

    out_ref[...] = jnp.dot(p_mat, payload_ref[...],
                           precision=jax.lax.Precision.HIGHEST,
                           preferred_element_type=f32)


def _mask_kernel(proto_ref, sel_ref, out_ref):
    f32 = jnp.float32
    c = pl.program_id(0)
    coeff = sel_ref[:, 8:40]
    z = jnp.dot(coeff, proto_ref[...], preferred_element_type=f32)
    m = jax.nn.sigmoid(z)
    m = jnp.where(m >= _MASK_T, m, 0.0)

    ratio_inv = jnp.asarray(128.0 / _IN_RES, f32)
    bb0 = sel_ref[:, 0:1] * ratio_inv
    bb1 = sel_ref[:, 1:2] * ratio_inv
    bb2 = sel_ref[:, 2:3] * ratio_inv
    bb3 = sel_ref[:, 3:4] * ratio_inv
    vf = sel_ref[:, 4:5] > 0.0

    g = c * _PBLK + jax.lax.broadcasted_iota(jnp.int32, (1, _PBLK), 1)
    rows_f = (g // 128).astype(f32)
    cols_f = (g % 128).astype(f32)
    ok = ((bb0 <= rows_f) & (rows_f <= bb2) &
          (bb1 <= cols_f) & (cols_f <= bb3) & vf)
    t = jnp.where(ok, m, 0.0)
    out_ref[...] = jnp.sum(t, axis=0, keepdims=True)


def kernel(boxes, proto):
    f32 = boxes.dtype
    b = boxes[0]
    box = b[:4]
    prob = b[4:8]
    coeff = b[8:]
    scores = jnp.max(prob, axis=0)
    cls = jnp.argmax(prob, axis=0)

    bt = box.T
    xyxy = jnp.concatenate([bt[:, :2] - bt[:, 2:] / 2,
                            bt[:, :2] + bt[:, 2:] / 2], axis=-1)
    b_nms = xyxy[:, jnp.array([1, 0, 3, 2])]

    order = jnp.argsort(-scores)
    payload = jnp.concatenate(
        [b_nms, scores[:, None], cls.astype(f32)[:, None],
         jnp.zeros((_N, 2), f32), coeff.T], axis=1)
    payload_s = payload[order]

    sel = pl.pallas_call(
        _nms_select_kernel,
        out_shape=jax.ShapeDtypeStruct((_SLOTS, 40), f32),
        compiler_params=pltpu.CompilerParams(
            vmem_limit_bytes=48 * 1024 * 1024),
    )(payload_s)

    proto_t = proto[0].reshape(_NPIX, 32).T
    masks_flat = pl.pallas_call(
        _mask_kernel,
        grid=(_NPIX // _PBLK,),
        in_specs=[
            pl.BlockSpec((32, _PBLK), lambda c: (0, c)),
            pl.BlockSpec((_SLOTS, 40), lambda c: (0, 0)),
        ],
        out_specs=pl.BlockSpec((1, _PBLK), lambda c: (0, c)),
        out_shape=jax.ShapeDtypeStruct((1, _NPIX), f32),
        compiler_params=pltpu.CompilerParams(
            dimension_semantics=("parallel",),
            vmem_limit_bytes=48 * 1024 * 1024),
    )(proto_t, sel)
    masks = masks_flat.reshape(128, 128)

    return (sel[:_MAXD, 4:5], sel[:_MAXD, 5:6], sel[:_MAXD, 0:4], masks)
